# baseline (speedup 1.0000x reference)
import jax
import jax.numpy as jnp
from jax import lax
from jax.experimental import pallas as pl
from jax.experimental.pallas import tpu as pltpu

_B, _N, _INF, _OUTF = 4, 1024, 256, 256
_PB = 2
_K = 256
_T = _N // _K


def _gat_body(inp_ref, adj_ref, w_ref, a_ref, out_ref, m_s, at_s, aa_s, sem):
    first = pl.program_id(0) == 0

    @pl.when(first)
    def _():
        for t in range(_T):
            pltpu.make_async_copy(adj_ref.at[pl.ds(t * _K, _K)],
                                  m_s.at[pl.ds(t * _K, _K)],
                                  sem.at[t]).start()
        pltpu.make_async_copy(a_ref, aa_s, sem.at[_T]).start()

    row = lax.broadcasted_iota(jnp.int32, (_N, 1), 0)
    w = w_ref[...]
    hs, hps = [], []
    for u in range(_PB):
        h = jnp.dot(inp_ref[u], w, preferred_element_type=jnp.float32)
        h = jnp.where(row == 0, 0.0, h)
        hp = pltpu.roll(h, 1, 0)
        hps.append(jnp.where(row == 0, 0.0, hp))
        hs.append(h)

    @pl.when(first)
    def _():
        g0 = jnp.zeros((_N, _OUTF), jnp.float32)
        for t in range(_T):
            pltpu.make_async_copy(adj_ref.at[pl.ds(t * _K, _K)],
                                  m_s.at[pl.ds(t * _K, _K)],
                                  sem.at[t]).wait()
            mt = (m_s[pl.ds(t * _K, _K), :] > 0).astype(jnp.float32)
            m_s[pl.ds(t * _K, _K), :] = mt
            g0 = g0 + lax.dot_general(
                mt, lax.slice(hps[0], (t * _K, 0), ((t + 1) * _K, _OUTF)),
                (((0,), (0,)), ((), ())),
                preferred_element_type=jnp.float32)
        g_s0 = g0
        pltpu.make_async_copy(a_ref, aa_s, sem.at[_T]).wait()
        at_s[...] = jnp.transpose(aa_s[...])
        _emit(hs[0], g_s0, at_s, row, out_ref, 0)

    @pl.when(jnp.logical_not(first))
    def _():
        g0 = lax.dot_general(m_s[...], hps[0], (((0,), (0,)), ((), ())),
                             preferred_element_type=jnp.float32)
        _emit(hs[0], g0, at_s, row, out_ref, 0)

    g1 = lax.dot_general(m_s[...], hps[1], (((0,), (0,)), ((), ())),
                         preferred_element_type=jnp.float32)
    _emit(hs[1], g1, at_s, row, out_ref, 1)


def _emit(h, g, at_s, row, out_ref, u):
    at = at_s[...]
    vals = (jnp.sum(h * at[:, :_OUTF], axis=1, keepdims=True)
            + jnp.sum(g * at[:, _OUTF:], axis=1, keepdims=True))
    vals = jnp.where(row == 0, 0.0, vals)
    ssum = jnp.sum(h, axis=0, keepdims=True)
    o = vals * ssum
    out_ref[u] = jnp.maximum(o, 0.2 * o)


def kernel(inp, adj, W, a):
    return pl.pallas_call(
        _gat_body,
        grid=(_B // _PB,),
        in_specs=[
            pl.BlockSpec((_PB, _N, _INF), lambda b: (b, 0, 0)),
            pl.BlockSpec(memory_space=pltpu.MemorySpace.HBM),
            pl.BlockSpec((_INF, _OUTF), lambda b: (0, 0)),
            pl.BlockSpec(memory_space=pltpu.MemorySpace.HBM),
        ],
        out_specs=pl.BlockSpec((_PB, _N, _OUTF), lambda b: (b, 0, 0)),
        out_shape=jax.ShapeDtypeStruct((_B, _N, _OUTF), jnp.float32),
        scratch_shapes=[
            pltpu.VMEM((_N, _N), jnp.float32),
            pltpu.VMEM((_N, 2 * _OUTF), jnp.float32),
            pltpu.VMEM((2 * _OUTF, _N), jnp.float32),
            pltpu.SemaphoreType.DMA((_T + 1,)),
        ],
        compiler_params=pltpu.CompilerParams(
            dimension_semantics=("arbitrary",),
        ),
    )(inp, adj, W, a)
